# baseline (speedup 1.0000x reference)
import functools

import jax
import jax.numpy as jnp
import numpy as np
from jax import lax
from jax.experimental import pallas as pl
from jax.experimental.pallas import tpu as pltpu
from jax.experimental.pallas import tpu_sc as plsc

N_NODES = 50000
N_EDGES = 800000
N_HOPS = 5
NC = 2
NS = 16
NW = NC * NS
CHUNK = 256
NBUF = 3
CPT = 198
EPAD = NS * CPT * CHUNK
RPT = N_NODES // NS
RSTAGE = CHUNK
NSTAGE = RPT // RSTAGE
RREM = RPT - NSTAGE * RSTAGE
F = 32
HT = NC * N_NODES

_BCAST_DNUMS = lax.GatherDimensionNumbers(
    offset_dims=(), collapsed_slice_dims=(0,), start_index_map=(0,))


def _bcast_lane(v16, idx):
    return lax.gather(v16, idx, _BCAST_DNUMS, (1,),
                      mode=lax.GatherScatterMode.PROMISE_IN_BOUNDS)


def _mlp_prologue(x, w, b, d1, d2):
    n, k = x.shape
    r = 2000
    g = n // r

    def body(x_ref, w_ref, b_ref, d1_ref, d2_ref, h_ref, dh_ref):
        acts = jnp.dot(x_ref[...], w_ref[...],
                       preferred_element_type=jnp.float32)
        acts = jnp.maximum(acts + b_ref[...], 0.0)
        dh = 0.5 * (d1_ref[...] + d2_ref[...]) * acts
        h_ref[0] = acts[:, 0:F]
        h_ref[1] = acts[:, F:2 * F]
        dh_ref[0] = dh[:, 0:F]
        dh_ref[1] = dh[:, F:2 * F]

    h, dh = pl.pallas_call(
        body,
        grid=(g,),
        in_specs=[
            pl.BlockSpec((r, k), lambda i: (i, 0)),
            pl.BlockSpec((k, 2 * F), lambda i: (0, 0)),
            pl.BlockSpec((1, 2 * F), lambda i: (0, 0)),
            pl.BlockSpec((r, 1), lambda i: (i, 0)),
            pl.BlockSpec((r, 1), lambda i: (i, 0)),
        ],
        out_specs=[
            pl.BlockSpec((NC, r, F), lambda i: (0, i, 0)),
            pl.BlockSpec((NC, r, F), lambda i: (0, i, 0)),
        ],
        out_shape=[
            jax.ShapeDtypeStruct((NC, n, F), jnp.float32),
            jax.ShapeDtypeStruct((NC, n, F), jnp.float32),
        ],
    )(x, w, b.reshape(1, 2 * F), d1, d2)
    return h.reshape(HT, F), dh.reshape(HT, F)


def _final_projection(ha, w2, b2):
    r = 2000
    g = N_NODES // r

    def body(lo_ref, hi_ref, w_ref, b_ref, o_ref):
        o_ref[...] = (
            jnp.dot(lo_ref[...], w_ref[0:F, :],
                    preferred_element_type=jnp.float32)
            + jnp.dot(hi_ref[...], w_ref[F:2 * F, :],
                      preferred_element_type=jnp.float32)
            + b_ref[...]
        )

    return pl.pallas_call(
        body,
        grid=(g,),
        in_specs=[
            pl.BlockSpec((r, F), lambda i: (i, 0)),
            pl.BlockSpec((r, F), lambda i: (i + g, 0)),
            pl.BlockSpec((2 * F, 16), lambda i: (0, 0)),
            pl.BlockSpec((1, 16), lambda i: (0, 0)),
        ],
        out_specs=pl.BlockSpec((r, 16), lambda i: (i, 0)),
        out_shape=jax.ShapeDtypeStruct((N_NODES, 16), jnp.float32),
    )(ha, ha, w2, b2.reshape(1, 16))


def _sc_body(ha0, hb0, da, db,
             src_aa, dst_aa, val_aa, src_ab, dst_ab, val_ab,
             src_ba, dst_ba, val_ba, src_bb, dst_bb, val_bb,
             ha_out, hb_out,
             acc, src0, src1, src2, dst0, dst1, dst2, val0, val1, val2,
             rows0, rows1, rows2,
             isem0, isem1, isem2, gsem0, gsem1, gsem2):
    c = lax.axis_index("c")
    s = lax.axis_index("s")
    zeros16 = lax.iota(jnp.int32, 16) * 0
    bcast_idx = [(zeros16 + e)[:, None] for e in range(16)]
    coff = c * N_NODES
    rbase = s * RPT
    hbase = coff + rbase
    srcb = (src0, src1, src2)
    dstb = (dst0, dst1, dst2)
    valb = (val0, val1, val2)
    rowsb = (rows0, rows1, rows2)
    isem = (isem0, isem1, isem2)
    gsem = (gsem0, gsem1, gsem2)

    def staged_copy(read, write):
        @pl.loop(0, NSTAGE)
        def _(z):
            pltpu.sync_copy(read(z * RSTAGE, RSTAGE), rows0)
            pltpu.sync_copy(rows0, write(z * RSTAGE, RSTAGE))
        rem = NSTAGE * RSTAGE
        pltpu.sync_copy(read(rem, RREM), rows0.at[pl.ds(0, RREM)])
        pltpu.sync_copy(rows0.at[pl.ds(0, RREM)], write(rem, RREM))

    def h_slice(ref):
        return lambda off, n: ref.at[pl.ds(hbase + off, n)]

    def acc_slice(off, n):
        return acc.at[pl.ds(rbase + off, n)]

    staged_copy(h_slice(ha0), h_slice(ha_out))
    staged_copy(h_slice(hb0), h_slice(hb_out))
    plsc.subcore_barrier()

    def accumulate(src_e, dst_e, val_e, h_src):
        def eslice(ref, k):
            return ref.at[s, pl.ds(k * CHUNK, CHUNK)]

        def stage(k, u):
            pltpu.async_copy(eslice(src_e, k), srcb[u], isem[u])
            pltpu.async_copy(eslice(dst_e, k), dstb[u], isem[u])
            pltpu.async_copy(eslice(val_e, k), valb[u], isem[u])

        def wait_stage(k, u):
            pltpu.make_async_copy(eslice(src_e, k), srcb[u], isem[u]).wait()
            pltpu.make_async_copy(eslice(dst_e, k), dstb[u], isem[u]).wait()
            pltpu.make_async_copy(eslice(val_e, k), valb[u], isem[u]).wait()

        def offadd(u):
            for g in range(CHUNK // 16):
                sl = pl.ds(g * 16, 16)
                srcb[u][sl] = srcb[u][sl] + coff

        def scale(u):
            @pl.loop(0, CHUNK // 16)
            def _scale(g16):
                v16 = valb[u][pl.ds(g16 * 16, 16)] * 0.5
                for e in range(16):
                    vb = _bcast_lane(v16, bcast_idx[e])
                    row = g16 * 16 + e
                    rowsb[u][row, pl.ds(0, 16)] = (
                        rowsb[u][row, pl.ds(0, 16)] * vb)
                    rowsb[u][row, pl.ds(16, 16)] = (
                        rowsb[u][row, pl.ds(16, 16)] * vb)

        def scatter(u):
            pltpu.sync_copy(rowsb[u], acc.at[dstb[u]], add=True)

        def gwait(u):
            pltpu.make_async_copy(h_src.at[srcb[u]], rowsb[u],
                                  gsem[u]).wait()

        def gissue(u):
            pltpu.async_copy(h_src.at[srcb[u]], rowsb[u], gsem[u])

        for u in range(NBUF):
            stage(u, u)
        for u in range(2):
            wait_stage(u, u)
            offadd(u)
            gissue(u)

        @pl.loop(0, (CPT - NBUF) // NBUF)
        def _trip(p):
            for u in range(NBUF):
                k = NBUF * p + u
                n2 = (u + 2) % NBUF
                wait_stage(k + 2, n2)
                offadd(n2)
                gissue(n2)
                gwait(u)
                scale(u)
                scatter(u)
                stage(k + NBUF, u)

        wait_stage(CPT - 1, (CPT - 1) % NBUF)
        offadd((CPT - 1) % NBUF)
        gissue((CPT - 1) % NBUF)
        for k in range(CPT - NBUF, CPT):
            u = k % NBUF
            gwait(u)
            scale(u)
            scatter(u)

    def phase(d_hbm, sets, h_dst):
        staged_copy(h_slice(d_hbm), acc_slice)
        plsc.subcore_barrier()
        for args in sets:
            accumulate(*args)
        plsc.subcore_barrier()
        staged_copy(acc_slice, h_slice(h_dst))
        plsc.subcore_barrier()

    @pl.loop(0, N_HOPS)
    def _hop(h):
        phase(da, [(src_aa, dst_aa, val_aa, ha_out),
                   (src_ab, dst_ab, val_ab, hb_out)], ha_out)
        phase(db, [(src_ba, dst_ba, val_ba, ha_out),
                   (src_bb, dst_bb, val_bb, hb_out)], hb_out)


@functools.cache
def _sc_hops():
    return pl.kernel(
        _sc_body,
        out_type=(jax.ShapeDtypeStruct((HT, F), jnp.float32),
                  jax.ShapeDtypeStruct((HT, F), jnp.float32)),
        mesh=plsc.VectorSubcoreMesh(core_axis_name="c", subcore_axis_name="s",
                                    num_cores=NC, num_subcores=NS),
        compiler_params=pltpu.CompilerParams(use_tc_tiling_on_sc=False),
        scratch_types=(
            [pltpu.VMEM_SHARED((N_NODES, F), jnp.float32)]
            + [pltpu.VMEM((CHUNK,), jnp.int32)] * NBUF
            + [pltpu.VMEM((CHUNK,), jnp.int32)] * NBUF
            + [pltpu.VMEM((CHUNK,), jnp.float32)] * NBUF
            + [pltpu.VMEM((CHUNK, F), jnp.float32)] * NBUF
            + [pltpu.SemaphoreType.DMA] * NBUF
            + [pltpu.SemaphoreType.DMA] * NBUF
        ),
    )


def _prep_edges(ei, vals):
    pad = EPAD - N_EDGES
    src = jnp.concatenate(
        [ei[1].astype(jnp.int32), jnp.zeros((pad,), jnp.int32)]
    ).reshape(NS, CPT * CHUNK)
    dst = jnp.concatenate(
        [ei[0].astype(jnp.int32), jnp.zeros((pad,), jnp.int32)]
    ).reshape(NS, CPT * CHUNK)
    val = jnp.concatenate(
        [vals, jnp.zeros((pad,), jnp.float32)]
    ).reshape(NS, CPT * CHUNK)
    return src, dst, val


def kernel(x_a, x_b, edge_index_aa, values_aa, edge_index_ab, values_ab,
           edge_index_ba, values_ba, edge_index_bb, values_bb,
           d_aa, d_ab, d_ba, d_bb, W1_a, b1_a, W1_b, b1_b, W2, b2):
    ha0, da = _mlp_prologue(x_a, W1_a, b1_a, d_aa, d_ab)
    hb0, db = _mlp_prologue(x_b, W1_b, b1_b, d_ba, d_bb)
    e_aa = _prep_edges(edge_index_aa, values_aa)
    e_ab = _prep_edges(edge_index_ab, values_ab)
    e_ba = _prep_edges(edge_index_ba, values_ba)
    e_bb = _prep_edges(edge_index_bb, values_bb)
    ha_fin, _ = _sc_hops()(ha0, hb0, da, db, *e_aa, *e_ab, *e_ba, *e_bb)
    return _final_projection(ha_fin, W2, b2)


# speedup vs baseline: 1.1233x; 1.1233x over previous
import functools

import jax
import jax.numpy as jnp
import numpy as np
from jax import lax
from jax.experimental import pallas as pl
from jax.experimental.pallas import tpu as pltpu
from jax.experimental.pallas import tpu_sc as plsc

N_NODES = 50000
N_EDGES = 800000
N_HOPS = 5
NC = 2
NS = 16
NW = NC * NS
CHUNK = 512
CPT = -(-N_EDGES // (NS * CHUNK))
NSUP = 14
CPS = CPT // NSUP
EPAD = NS * CPT * CHUNK
RPT = N_NODES // NS
RSTAGE = CHUNK
NSTAGE = RPT // RSTAGE
RREM = RPT - NSTAGE * RSTAGE
F = 32
HT = NC * N_NODES

_BCAST_DNUMS = lax.GatherDimensionNumbers(
    offset_dims=(), collapsed_slice_dims=(0,), start_index_map=(0,))


def _bcast_lane(v16, idx):
    return lax.gather(v16, idx, _BCAST_DNUMS, (1,),
                      mode=lax.GatherScatterMode.PROMISE_IN_BOUNDS)


def _mlp_prologue(x, w, b, d1, d2):
    n, k = x.shape
    r = 2000
    g = n // r

    def body(x_ref, w_ref, b_ref, d1_ref, d2_ref, h_ref, dh_ref):
        acts = jnp.dot(x_ref[...], w_ref[...],
                       preferred_element_type=jnp.float32)
        acts = jnp.maximum(acts + b_ref[...], 0.0)
        dh = 0.5 * (d1_ref[...] + d2_ref[...]) * acts
        h_ref[0] = acts[:, 0:F]
        h_ref[1] = acts[:, F:2 * F]
        dh_ref[0] = dh[:, 0:F]
        dh_ref[1] = dh[:, F:2 * F]

    h, dh = pl.pallas_call(
        body,
        grid=(g,),
        in_specs=[
            pl.BlockSpec((r, k), lambda i: (i, 0)),
            pl.BlockSpec((k, 2 * F), lambda i: (0, 0)),
            pl.BlockSpec((1, 2 * F), lambda i: (0, 0)),
            pl.BlockSpec((r, 1), lambda i: (i, 0)),
            pl.BlockSpec((r, 1), lambda i: (i, 0)),
        ],
        out_specs=[
            pl.BlockSpec((NC, r, F), lambda i: (0, i, 0)),
            pl.BlockSpec((NC, r, F), lambda i: (0, i, 0)),
        ],
        out_shape=[
            jax.ShapeDtypeStruct((NC, n, F), jnp.float32),
            jax.ShapeDtypeStruct((NC, n, F), jnp.float32),
        ],
    )(x, w, b.reshape(1, 2 * F), d1, d2)
    return h.reshape(HT, F), dh.reshape(HT, F)


def _final_projection(ha, w2, b2):
    r = 2000
    g = N_NODES // r

    def body(lo_ref, hi_ref, w_ref, b_ref, o_ref):
        o_ref[...] = (
            jnp.dot(lo_ref[...], w_ref[0:F, :],
                    preferred_element_type=jnp.float32)
            + jnp.dot(hi_ref[...], w_ref[F:2 * F, :],
                      preferred_element_type=jnp.float32)
            + b_ref[...]
        )

    return pl.pallas_call(
        body,
        grid=(g,),
        in_specs=[
            pl.BlockSpec((r, F), lambda i: (i, 0)),
            pl.BlockSpec((r, F), lambda i: (i + g, 0)),
            pl.BlockSpec((2 * F, 16), lambda i: (0, 0)),
            pl.BlockSpec((1, 16), lambda i: (0, 0)),
        ],
        out_specs=pl.BlockSpec((r, 16), lambda i: (i, 0)),
        out_shape=jax.ShapeDtypeStruct((N_NODES, 16), jnp.float32),
    )(ha, ha, w2, b2.reshape(1, 16))


def _sc_body(ha0, hb0, da, db,
             src_aa, dst_aa, val_aa, src_ab, dst_ab, val_ab,
             src_ba, dst_ba, val_ba, src_bb, dst_bb, val_bb,
             ha_out, hb_out,
             acc, src_v, dst_v, vals_v, rows_v):
    c = lax.axis_index("c")
    s = lax.axis_index("s")
    zeros16 = lax.iota(jnp.int32, 16) * 0
    bcast_idx = [(zeros16 + e)[:, None] for e in range(16)]
    coff = c * N_NODES
    rbase = s * RPT
    hbase = coff + rbase
    def staged_copy(read, write):
        @pl.loop(0, NSTAGE)
        def _(z):
            pltpu.sync_copy(read(z * RSTAGE, RSTAGE), rows_v)
            pltpu.sync_copy(rows_v, write(z * RSTAGE, RSTAGE))
        rem = NSTAGE * RSTAGE
        pltpu.sync_copy(read(rem, RREM), rows_v.at[pl.ds(0, RREM)])
        pltpu.sync_copy(rows_v.at[pl.ds(0, RREM)], write(rem, RREM))

    def h_slice(ref):
        return lambda off, n: ref.at[pl.ds(hbase + off, n)]

    def acc_slice(off, n):
        return acc.at[pl.ds(rbase + off, n)]

    staged_copy(h_slice(ha0), h_slice(ha_out))
    staged_copy(h_slice(hb0), h_slice(hb_out))
    plsc.subcore_barrier()

    def accumulate(src_e, dst_e, val_e, h_src):
        @pl.loop(0, NSUP)
        def _super(sp):
            sup = CPS * CHUNK
            pltpu.sync_copy(src_e.at[s, pl.ds(sp * sup, sup)], src_v)
            pltpu.sync_copy(dst_e.at[s, pl.ds(sp * sup, sup)], dst_v)
            pltpu.sync_copy(val_e.at[s, pl.ds(sp * sup, sup)], vals_v)

            @pl.loop(0, CPS)
            def _chunk(j):
                base = j * CHUNK

                @pl.loop(0, CHUNK // 128)
                def _off(g8):
                    for u in range(8):
                        sl = pl.ds(base + g8 * 128 + u * 16, 16)
                        src_v[sl] = src_v[sl] + coff

                pltpu.sync_copy(h_src.at[src_v.at[pl.ds(base, CHUNK)]],
                                rows_v)

                @pl.loop(0, CHUNK // 16)
                def _scale(g16):
                    v16 = vals_v[pl.ds(base + g16 * 16, 16)] * 0.5
                    for e in range(16):
                        vb = _bcast_lane(v16, bcast_idx[e])
                        row = g16 * 16 + e
                        rows_v[row, pl.ds(0, 16)] = (
                            rows_v[row, pl.ds(0, 16)] * vb)
                        rows_v[row, pl.ds(16, 16)] = (
                            rows_v[row, pl.ds(16, 16)] * vb)

                pltpu.sync_copy(rows_v, acc.at[dst_v.at[pl.ds(base, CHUNK)]],
                                add=True)

    def phase(d_hbm, sets, h_dst):
        staged_copy(h_slice(d_hbm), acc_slice)
        plsc.subcore_barrier()
        for args in sets:
            accumulate(*args)
        plsc.subcore_barrier()
        staged_copy(acc_slice, h_slice(h_dst))
        plsc.subcore_barrier()

    @pl.loop(0, N_HOPS)
    def _hop(h):
        phase(da, [(src_aa, dst_aa, val_aa, ha_out),
                   (src_ab, dst_ab, val_ab, hb_out)], ha_out)
        phase(db, [(src_ba, dst_ba, val_ba, ha_out),
                   (src_bb, dst_bb, val_bb, hb_out)], hb_out)


@functools.cache
def _sc_hops():
    return pl.kernel(
        _sc_body,
        out_type=(jax.ShapeDtypeStruct((HT, F), jnp.float32),
                  jax.ShapeDtypeStruct((HT, F), jnp.float32)),
        mesh=plsc.VectorSubcoreMesh(core_axis_name="c", subcore_axis_name="s",
                                    num_cores=NC, num_subcores=NS),
        compiler_params=pltpu.CompilerParams(use_tc_tiling_on_sc=False),
        scratch_types=[
            pltpu.VMEM_SHARED((N_NODES, F), jnp.float32),
            pltpu.VMEM((CPS * CHUNK,), jnp.int32),
            pltpu.VMEM((CPS * CHUNK,), jnp.int32),
            pltpu.VMEM((CPS * CHUNK,), jnp.float32),
            pltpu.VMEM((CHUNK, F), jnp.float32),
        ],
    )


def _prep_edges(ei, vals):
    pad = EPAD - N_EDGES
    src = jnp.concatenate(
        [ei[1].astype(jnp.int32), jnp.zeros((pad,), jnp.int32)]
    ).reshape(NS, CPT * CHUNK)
    dst = jnp.concatenate(
        [ei[0].astype(jnp.int32), jnp.zeros((pad,), jnp.int32)]
    ).reshape(NS, CPT * CHUNK)
    val = jnp.concatenate(
        [vals, jnp.zeros((pad,), jnp.float32)]
    ).reshape(NS, CPT * CHUNK)
    return src, dst, val


def kernel(x_a, x_b, edge_index_aa, values_aa, edge_index_ab, values_ab,
           edge_index_ba, values_ba, edge_index_bb, values_bb,
           d_aa, d_ab, d_ba, d_bb, W1_a, b1_a, W1_b, b1_b, W2, b2):
    ha0, da = _mlp_prologue(x_a, W1_a, b1_a, d_aa, d_ab)
    hb0, db = _mlp_prologue(x_b, W1_b, b1_b, d_ba, d_bb)
    e_aa = _prep_edges(edge_index_aa, values_aa)
    e_ab = _prep_edges(edge_index_ab, values_ab)
    e_ba = _prep_edges(edge_index_ba, values_ba)
    e_bb = _prep_edges(edge_index_bb, values_bb)
    ha_fin, _ = _sc_hops()(ha0, hb0, da, db, *e_aa, *e_ab, *e_ba, *e_bb)
    return _final_projection(ha_fin, W2, b2)
